# Initial kernel scaffold; baseline (speedup 1.0000x reference)
import jax
import jax.numpy as jnp
from jax.experimental import pallas as pl


def kernel(des, tweet, num_prop, cat_prop, edge_index, edge_type, W_des, b_des, W_tweet, b_tweet, W_num, b_num, W_cat, b_cat, W_in, b_in, W_rel, W_root, b_rgcn, W_gate, We1, be1, We2, be2):
    raise NotImplementedError("write your pallas kernel here")


# speedup vs baseline: 3.7301x; 3.7301x over previous
import functools

import jax
import jax.numpy as jnp
from jax import lax
from jax.experimental import pallas as pl
from jax.experimental.pallas import tpu as pltpu
from jax.experimental.pallas import tpu_sc as plsc

N = 10000
E = 160000
D = 256
NEXP = 8
OUT = 128
TM = 2000
NROWS = N + 112
TRASH = N
EPW = E // 16
NCH = 79
EPAD = NCH * 128

_SELU_SCALE = 1.0507009873554805
_SELU_ALPHA = 1.6732632423543772


def _selu(x):
    safe = jnp.where(x > 0, 0.0, x)
    return _SELU_SCALE * jnp.where(x > 0, x, _SELU_ALPHA * jnp.expm1(safe))


def _tabs_body(x, wr, out):
    y = jnp.dot(x[...], wr[0], preferred_element_type=jnp.float32)
    out[:, :128] = y
    col = lax.broadcasted_iota(jnp.int32, (TM, 16), 1)
    out[:, 128:144] = jnp.where(col == 0, 1.0, 0.0)


def _tabs(x, W_rel):
    return pl.pallas_call(
        _tabs_body,
        grid=(N // TM, 4),
        in_specs=[
            pl.BlockSpec((TM, 256), lambda i, v: (i, 0)),
            pl.BlockSpec((1, 256, 128), lambda i, v: (v // 2, 0, v % 2)),
        ],
        out_specs=pl.BlockSpec((TM, 144), lambda i, v: (v * (N // TM) + i, 0)),
        out_shape=jax.ShapeDtypeStruct((4 * N, 144), jnp.float32),
    )(x, W_rel)


G = 64
NCH2 = EPAD // G
_SENT = 3 << 28


def _agg_body(tabs, epk, zeros, out0, out1, e_v, gidx, didx, gbuf, acc, sem0, sem1):
    c = lax.axis_index("c")
    s = lax.axis_index("s")
    ebase = s * EPW

    pltpu.sync_copy(epk.at[pl.ds(ebase, EPW)], e_v.at[pl.ds(0, EPW)])
    for j in range((EPAD - EPW) // 16):
        e_v[pl.ds(EPW + j * 16, 16)] = jnp.full((16,), _SENT, jnp.int32)

    rpw = NROWS // 16
    msk = jnp.full((16,), (1 << 14) - 1, jnp.int32)
    sems = (sem0, sem1)

    for r in range(2):
        pltpu.sync_copy(zeros.at[pl.ds(s * rpw, rpw)], acc.at[pl.ds(s * rpw, rpw)])
        plsc.subcore_barrier()
        vbase = (2 * r) * N + c * N

        def fill(ci, b):
            for jj in range(G // 16):
                off = ci * G + jj * 16
                e = e_v[pl.ds(off, 16)]
                gidx[b, pl.ds(jj * 16, 16)] = (e & msk) + vbase
                didx[b, pl.ds(jj * 16, 16)] = jnp.where(
                    (e >> 28) == r, (e >> 14) & msk, TRASH)
            return pltpu.async_copy(tabs.at[gidx.at[b]], gbuf.at[b], sems[b])

        def outer(gi, _):
            h0 = fill(gi * 2, 0)
            h1 = fill(gi * 2 + 1, 1)
            h0.wait()
            pltpu.sync_copy(gbuf.at[0], acc.at[didx.at[0]], add=True)
            h1.wait()
            pltpu.sync_copy(gbuf.at[1], acc.at[didx.at[1]], add=True)
            return _

        lax.fori_loop(0, NCH2 // 2, outer, 0)
        plsc.subcore_barrier()
        out = out0 if r == 0 else out1
        pltpu.sync_copy(acc.at[pl.ds(s * rpw, rpw)],
                        out.at[c, pl.ds(s * rpw, rpw)])
        plsc.subcore_barrier()


def _aggregate(tabs, epk, zeros):
    kern = functools.partial(
        pl.kernel,
        out_type=[jax.ShapeDtypeStruct((2, NROWS, 144), jnp.float32)] * 2,
        mesh=plsc.VectorSubcoreMesh(core_axis_name="c", subcore_axis_name="s"),
        compiler_params=pltpu.CompilerParams(use_tc_tiling_on_sc=False),
        scratch_types=[
            pltpu.VMEM((EPAD,), jnp.int32),
            pltpu.VMEM((2, G), jnp.int32),
            pltpu.VMEM((2, G), jnp.int32),
            pltpu.VMEM((2, G, 144), jnp.float32),
            pltpu.VMEM_SHARED((NROWS, 144), jnp.float32),
            pltpu.SemaphoreType.DMA,
            pltpu.SemaphoreType.DMA,
        ],
    )
    return kern(_agg_body)(tabs, epk, zeros)


def _combine_body(x, wroot, b, a00, a01, a10, a11, out):
    xn = jnp.dot(x[...], wroot[...], preferred_element_type=jnp.float32) + b[...]
    parts = []
    for h, (ar0, ar1) in enumerate(((a00, a10), (a01, a11))):
        p = xn[:, h * 128:(h + 1) * 128]
        for a in (ar0, ar1):
            cnt = jnp.clip(a[0, :, 128:129], 1.0, None)
            p = p + a[0, :, :128] / cnt
        parts.append(p)
    out[...] = jnp.concatenate(parts, axis=1)


def _combine(x, W_root, b_rgcn, acc0, acc1):
    full = lambda shp: pl.BlockSpec(shp, lambda i: (0,) * len(shp))
    aspec = lambda h: pl.BlockSpec((1, TM, 144), lambda i: (h, i, 0))
    return pl.pallas_call(
        _combine_body,
        grid=(N // TM,),
        in_specs=[
            pl.BlockSpec((TM, 256), lambda i: (i, 0)),
            full((256, 256)), full((1, 256)),
            aspec(0), aspec(1), aspec(0), aspec(1),
        ],
        out_specs=pl.BlockSpec((TM, 256), lambda i: (i, 0)),
        out_shape=jax.ShapeDtypeStruct((N, 256), jnp.float32),
    )(x, W_root, b_rgcn, acc0, acc0, acc1, acc1)


def _moe_body(x, wg, we1, be1, we2, be2, out):
    xv = x[...]
    logits = jnp.dot(xv, wg[...], preferred_element_type=jnp.float32)
    iota = lax.broadcasted_iota(jnp.int32, (TM, NEXP), 1)
    m1 = jnp.max(logits, axis=1, keepdims=True)
    cand1 = jnp.where(logits == m1, iota, NEXP)
    i1 = jnp.min(cand1, axis=1, keepdims=True)
    mask1 = iota == i1
    masked = jnp.where(mask1, -jnp.inf, logits)
    m2 = jnp.max(masked, axis=1, keepdims=True)
    cand2 = jnp.where(masked == m2, iota, NEXP)
    i2 = jnp.min(cand2, axis=1, keepdims=True)
    mask2 = iota == i2
    e2 = jnp.exp(m2 - m1)
    g1 = 1.0 / (1.0 + e2)
    g2 = e2 * g1
    w = g1 * mask1.astype(jnp.float32) + g2 * mask2.astype(jnp.float32)
    acc_o = jnp.zeros((TM, OUT), jnp.float32)
    xb = xv.astype(jnp.bfloat16)
    for e in range(NEXP):
        h = jnp.dot(xb, we1[e].astype(jnp.bfloat16),
                    preferred_element_type=jnp.float32) + be1[e]
        h = jnp.where(h > 0, h, 0.01 * h)
        o = jnp.dot(h.astype(jnp.bfloat16), we2[e].astype(jnp.bfloat16),
                    preferred_element_type=jnp.float32) + be2[e]
        acc_o = acc_o + w[:, e:e + 1] * o
    out[...] = acc_o


def _moe(x, W_gate, We1, be1, We2, be2):
    full = lambda shp: pl.BlockSpec(shp, lambda i: (0,) * len(shp))
    return pl.pallas_call(
        _moe_body,
        grid=(N // TM,),
        in_specs=[
            pl.BlockSpec((TM, 256), lambda i: (i, 0)),
            full((256, NEXP)),
            full((NEXP, 256, 256)), full((NEXP, 1, 256)),
            full((NEXP, 256, OUT)), full((NEXP, 1, OUT)),
        ],
        out_specs=pl.BlockSpec((TM, OUT), lambda i: (i, 0)),
        out_shape=jax.ShapeDtypeStruct((N, OUT), jnp.float32),
    )(x, W_gate, We1, be1, We2, be2)


def kernel(des, tweet, num_prop, cat_prop, edge_index, edge_type,
           W_des, b_des, W_tweet, b_tweet, W_num, b_num, W_cat, b_cat,
           W_in, b_in, W_rel, W_root, b_rgcn,
           W_gate, We1, be1, We2, be2):
    f32 = jnp.float32
    src = edge_index[0].astype(jnp.int32)
    dst = edge_index[1].astype(jnp.int32)
    et = edge_type.astype(jnp.int32)
    epk = src | (dst << 14) | (et << 28)
    zeros = jnp.zeros((NROWS, 144), f32)

    d_ = jax.nn.selu(des @ W_des + b_des)
    t_ = jax.nn.selu(tweet @ W_tweet + b_tweet)
    n_ = jax.nn.selu(num_prop @ W_num + b_num)
    c_ = jax.nn.selu(cat_prop @ W_cat + b_cat)
    x = jax.nn.selu(jnp.concatenate((d_, t_, n_, c_), axis=1) @ W_in + b_in)
    for _ in range(2):
        tabs = _tabs(x, W_rel)
        acc0, acc1 = _aggregate(tabs, epk, zeros)
        x = _combine(x, W_root, b_rgcn.reshape(1, 256), acc0, acc1)
    return _moe(x, W_gate, We1.reshape(NEXP, 256, 256), be1.reshape(NEXP, 1, 256),
                We2.reshape(NEXP, 256, OUT), be2.reshape(NEXP, 1, OUT))
